# baseline (speedup 1.0000x reference)
import functools

import jax
import jax.numpy as jnp
from jax import lax
from jax.experimental import pallas as pl
from jax.experimental.pallas import tpu as pltpu
from jax.experimental.pallas import tpu_sc as plsc

DIM = 64
_info = plsc.get_sparse_core_info()
NC, NS = _info.num_cores, _info.num_subcores
NW = NC * NS

BLK = 128
SEQ_LEN = 200
BT = 4096 // BLK
N_BLOCKS = SEQ_LEN * BT
BLOCKS_PER_W = N_BLOCKS // NW


def _body(idx_hbm, tab_hbm, out_hbm,
          idx_all, pr0, pr1, rows0, rows1, blk0, blk1,
          sem_g, sem_s):
  wid = lax.axis_index("s") * NC + lax.axis_index("c")
  base_blk = wid * BLOCKS_PER_W
  base_idx = base_blk * BLK

  pltpu.sync_copy(idx_hbm.at[pl.ds(base_idx, BLOCKS_PER_W * BLK)], idx_all)

  iota16 = lax.iota(jnp.int32, 16)

  def compute_pr(k, pr_ref):
    for jg in range(8):
      v = idx_all[pl.ds(k * BLK + jg * 16, 16)]
      pr_ref[pl.ds(jg * 16, 16)] = jnp.right_shift(v, 1)

  def start_gather(pr_ref, rows_ref, slot):
    pltpu.make_async_copy(tab_hbm.at[pr_ref], rows_ref,
                          sem_g.at[slot]).start()

  def wait_gather(pr_ref, rows_ref, slot):
    pltpu.make_async_copy(tab_hbm.at[pr_ref], rows_ref,
                          sem_g.at[slot]).wait()

  def transpose(k, rows_ref, blk_ref):
    for jg in range(8):
      jvec = jnp.full((16,), jg * 16, jnp.int32) + iota16
      v = idx_all[pl.ds(k * BLK + jg * 16, 16)]
      colb = jnp.left_shift(jnp.bitwise_and(v, 1), 6)
      for d in range(DIM):
        col = colb + d
        g = plsc.load_gather(rows_ref, [jvec, col])
        blk_ref[pl.ds(d * BLK + jg * 16, 16)] = g

  def start_store(k, blk_ref, slot):
    g = base_blk + k
    s = lax.div(g, BT)
    bt = lax.rem(g, BT)
    for d8 in range(8):
      off = ((s * 8 + d8) * BT + bt) * (8 * BLK)
      pltpu.make_async_copy(blk_ref.at[pl.ds(d8 * 8 * BLK, 8 * BLK)],
                            out_hbm.at[pl.ds(off, 8 * BLK)],
                            sem_s.at[slot]).start()

  def wait_store(blk_ref, slot):
    for d8 in range(8):
      pltpu.make_async_copy(blk_ref.at[pl.ds(d8 * 8 * BLK, 8 * BLK)],
                            out_hbm.at[pl.ds(0, 8 * BLK)],
                            sem_s.at[slot]).wait()

  compute_pr(0, pr0)
  start_gather(pr0, rows0, 0)

  def pair_body(p, _):
    k0 = 2 * p

    compute_pr(k0 + 1, pr1)
    start_gather(pr1, rows1, 1)
    wait_gather(pr0, rows0, 0)

    @pl.when(p > 0)
    def _():
      wait_store(blk0, 0)

    transpose(k0, rows0, blk0)
    start_store(k0, blk0, 0)

    @pl.when(k0 + 2 < BLOCKS_PER_W)
    def _():
      compute_pr(k0 + 2, pr0)
      start_gather(pr0, rows0, 0)

    wait_gather(pr1, rows1, 1)

    @pl.when(p > 0)
    def _():
      wait_store(blk1, 1)

    transpose(k0 + 1, rows1, blk1)
    start_store(k0 + 1, blk1, 1)
    return 0

  lax.fori_loop(0, BLOCKS_PER_W // 2, pair_body, 0, unroll=False)
  wait_store(blk0, 0)
  wait_store(blk1, 1)


def kernel(indices, table):
  batch, seq = indices.shape
  n = batch * seq
  vocab = table.shape[0]

  idx_t = indices.T.reshape(n).astype(jnp.int32)
  tab2 = table.reshape(vocab // 2, 2 * DIM)

  mesh = plsc.VectorSubcoreMesh(core_axis_name="c", subcore_axis_name="s")
  k = functools.partial(
      pl.kernel,
      mesh=mesh,
      out_type=jax.ShapeDtypeStruct((n * DIM,), jnp.float32),
      scratch_types=[
          pltpu.VMEM((BLOCKS_PER_W * BLK,), jnp.int32),
          pltpu.VMEM((BLK,), jnp.int32),
          pltpu.VMEM((BLK,), jnp.int32),
          pltpu.VMEM((BLK, 2 * DIM), jnp.float32),
          pltpu.VMEM((BLK, 2 * DIM), jnp.float32),
          pltpu.VMEM((DIM * BLK,), jnp.float32),
          pltpu.VMEM((DIM * BLK,), jnp.float32),
          pltpu.SemaphoreType.DMA((2,)),
          pltpu.SemaphoreType.DMA((2,)),
      ],
      compiler_params=pltpu.CompilerParams(needs_layout_passes=False),
  )(_body)

  out_flat = k(idx_t, tab2)
  out5 = out_flat.reshape(seq, DIM // 8, batch // BLK, 8, BLK)
  return out5.transpose(2, 4, 0, 1, 3).reshape(batch, seq, DIM)
